# Initial kernel scaffold; baseline (speedup 1.0000x reference)
import jax
import jax.numpy as jnp
from jax.experimental import pallas as pl


def kernel(state, table, W1, b1, W2, b2, W3, b3):
    raise NotImplementedError("write your pallas kernel here")


# speedup vs baseline: 5.1643x; 5.1643x over previous
import functools

import jax
import jax.numpy as jnp
from jax import lax
from jax.experimental import pallas as pl
from jax.experimental.pallas import tpu as pltpu
from jax.experimental.pallas import tpu_sc as plsc

EMBED = 256
BATCH = 16384
N_IDX = 2 * BATCH
NC, NS = 2, 16
NW = NC * NS
ROWS_PER_W = N_IDX // NW
CHUNK = 128
NCHUNK = ROWS_PER_W // CHUNK
NBUF = 2


def _gather_body(idx_hbm, table_hbm, out_hbm, idx_v, rows0, rows1, g0, g1, w0, w1):
    wid = lax.axis_index("s") * NC + lax.axis_index("c")
    base = wid * ROWS_PER_W
    pltpu.sync_copy(idx_hbm.at[wid], idx_v)
    bufs = (rows0, rows1)
    gsems = (g0, g1)
    wsems = (w0, w1)
    gh = [None] * NCHUNK
    wh = [None] * NCHUNK

    gh[0] = pltpu.async_copy(table_hbm.at[idx_v.at[0]], bufs[0], gsems[0])
    for c in range(NCHUNK):
        b = c % NBUF
        if c + 1 < NCHUNK:
            nb = (c + 1) % NBUF
            if c + 1 >= NBUF:
                wh[c + 1 - NBUF].wait()
            gh[c + 1] = pltpu.async_copy(table_hbm.at[idx_v.at[c + 1]], bufs[nb], gsems[nb])
        gh[c].wait()
        wh[c] = pltpu.async_copy(bufs[b], out_hbm.at[pl.ds(base + c * CHUNK, CHUNK)], wsems[b])
    for c in range(NCHUNK - NBUF, NCHUNK):
        wh[c].wait()


_gather_rows = functools.partial(
    pl.kernel,
    out_type=jax.ShapeDtypeStruct((N_IDX, EMBED), jnp.float32),
    mesh=plsc.VectorSubcoreMesh(core_axis_name="c", subcore_axis_name="s"),
    scratch_types=[
        pltpu.VMEM((NCHUNK, CHUNK), jnp.int32),
        pltpu.VMEM((CHUNK, EMBED), jnp.float32),
        pltpu.VMEM((CHUNK, EMBED), jnp.float32),
        pltpu.SemaphoreType.DMA,
        pltpu.SemaphoreType.DMA,
        pltpu.SemaphoreType.DMA,
        pltpu.SemaphoreType.DMA,
    ],
)(_gather_body)


BLK = 2048


def _mlp_body(x_ref, w1_ref, b1_ref, w2_ref, b2_ref, w3_ref, b3_ref, o_ref):
    x = x_ref[...]
    h = jnp.dot(x, w1_ref[...], preferred_element_type=jnp.float32) + b1_ref[...]
    h = jnp.maximum(h, 0.0)
    h = jnp.dot(h, w2_ref[...], preferred_element_type=jnp.float32) + b2_ref[...]
    h = jnp.maximum(h, 0.0)
    o_ref[...] = jnp.dot(h, w3_ref[...], preferred_element_type=jnp.float32) + b3_ref[...]


def _mlp(x, W1, b1, W2, b2, W3, b3):
    grid = (BATCH // BLK,)
    full = lambda shape: pl.BlockSpec(shape, lambda i: (0, 0))
    return pl.pallas_call(
        _mlp_body,
        grid=grid,
        in_specs=[
            pl.BlockSpec((BLK, 512), lambda i: (i, 0)),
            full((512, 64)),
            full((1, 64)),
            full((64, 64)),
            full((1, 64)),
            full((64, 64)),
            full((1, 64)),
        ],
        out_specs=pl.BlockSpec((BLK, 64), lambda i: (i, 0)),
        out_shape=jax.ShapeDtypeStruct((BATCH, 64), jnp.float32),
    )(x, W1, b1, W2, b2, W3, b3)


def kernel(state, table, W1, b1, W2, b2, W3, b3):
    idx = state.reshape(NW, NCHUNK, CHUNK)
    rows = _gather_rows(idx, table)
    x = rows.reshape(BATCH, 2 * EMBED)
    return _mlp(x, W1, b1.reshape(1, 64), W2, b2.reshape(1, 64), W3, b3.reshape(1, 64))


# speedup vs baseline: 8.5719x; 1.6599x over previous
import functools

import jax
import jax.numpy as jnp
from jax import lax
from jax.experimental import pallas as pl
from jax.experimental.pallas import tpu as pltpu
from jax.experimental.pallas import tpu_sc as plsc

EMBED = 256
BATCH = 16384
NC, NS = 2, 16
NW = NC * NS
ROWS_PER_W = BATCH // NW
CHUNK = 128
NCHUNK = ROWS_PER_W // CHUNK
NBUF = 2


def _gather_body(idx_hbm, table_hbm, out_hbm, idx_v, rows0, rows1, g0, g1, w0, w1):
    wid = lax.axis_index("s") * NC + lax.axis_index("c")
    base = wid * ROWS_PER_W
    pltpu.sync_copy(idx_hbm.at[wid], idx_v)
    bufs = (rows0, rows1)
    gsems = (g0, g1)
    wsems = (w0, w1)
    NSTEP = NCHUNK * 2
    gh = [None] * NSTEP
    wh = [None] * NSTEP

    def dst(s):
        c, h = divmod(s, 2)
        return out_hbm.at[pl.ds(base + c * CHUNK, CHUNK), pl.ds(h * EMBED, EMBED)]

    def start_gather(s):
        c, h = divmod(s, 2)
        gh[s] = pltpu.async_copy(table_hbm.at[idx_v.at[c, h]], bufs[s % NBUF], gsems[s % NBUF])

    start_gather(0)
    for s in range(NSTEP):
        b = s % NBUF
        if s + 1 < NSTEP:
            if s + 1 >= NBUF:
                wh[s + 1 - NBUF].wait()
            start_gather(s + 1)
        gh[s].wait()
        wh[s] = pltpu.async_copy(bufs[b], dst(s), wsems[b])
    for s in range(NSTEP - NBUF, NSTEP):
        wh[s].wait()


_gather_cat = functools.partial(
    pl.kernel,
    out_type=jax.ShapeDtypeStruct((BATCH, 2 * EMBED), jnp.float32),
    mesh=plsc.VectorSubcoreMesh(core_axis_name="c", subcore_axis_name="s"),
    scratch_types=[
        pltpu.VMEM((NCHUNK, 2, CHUNK), jnp.int32),
        pltpu.VMEM((CHUNK, EMBED), jnp.float32),
        pltpu.VMEM((CHUNK, EMBED), jnp.float32),
        pltpu.SemaphoreType.DMA,
        pltpu.SemaphoreType.DMA,
        pltpu.SemaphoreType.DMA,
        pltpu.SemaphoreType.DMA,
    ],
)(_gather_body)


BLK = 2048


def _mlp_body(x_ref, w1_ref, b1_ref, w2_ref, b2_ref, w3_ref, b3_ref, o_ref):
    x = x_ref[...]
    h = jnp.dot(x, w1_ref[...], preferred_element_type=jnp.float32) + b1_ref[...]
    h = jnp.maximum(h, 0.0)
    h = jnp.dot(h, w2_ref[...], preferred_element_type=jnp.float32) + b2_ref[...]
    h = jnp.maximum(h, 0.0)
    o_ref[...] = jnp.dot(h, w3_ref[...], preferred_element_type=jnp.float32) + b3_ref[...]


def _mlp(x, W1, b1, W2, b2, W3, b3):
    grid = (BATCH // BLK,)
    full = lambda shape: pl.BlockSpec(shape, lambda i: (0, 0))
    return pl.pallas_call(
        _mlp_body,
        grid=grid,
        in_specs=[
            pl.BlockSpec((BLK, 512), lambda i: (i, 0)),
            full((512, 64)),
            full((1, 64)),
            full((64, 64)),
            full((1, 64)),
            full((64, 64)),
            full((1, 64)),
        ],
        out_specs=pl.BlockSpec((BLK, 64), lambda i: (i, 0)),
        out_shape=jax.ShapeDtypeStruct((BATCH, 64), jnp.float32),
    )(x, W1, b1, W2, b2, W3, b3)


def kernel(state, table, W1, b1, W2, b2, W3, b3):
    idx = state.reshape(NW, NCHUNK, CHUNK, 2).transpose(0, 1, 3, 2)
    x = _gather_cat(idx, table)
    return _mlp(x, W1, b1.reshape(1, 64), W2, b2.reshape(1, 64), W3, b3.reshape(1, 64))
